# baseline (speedup 1.0000x reference)
import functools

import jax
import jax.numpy as jnp
from jax import lax
from jax.experimental import pallas as pl
from jax.experimental.pallas import tpu as pltpu
from jax.experimental.pallas import tpu_sc as plsc

N_NODES = 50000
NPAD = 50048
EDGES = 800000
EP = 811008
LANES = 16
NCORES = 2
NSUB = 16
CHUNK = 96
NJ = 8
NUM_CLS = 40


def _segsum_sc(table_a, table_b, sd, d_eff, full_range):
    TR = sd.shape[0] // 2
    PT = TR // NSUB if full_range else TR // (NCORES * NSUB)
    NB = PT // NJ
    assert PT % NJ == 0
    PAIRS, TAIL = NB // 2, NB % 2
    stripe = NPAD // NSUB
    ZFULL, ZTAIL = stripe // CHUNK, stripe % CHUNK
    assert ZTAIL % 8 == 0
    mesh = plsc.VectorSubcoreMesh(core_axis_name="c", subcore_axis_name="s")

    @functools.partial(
        pl.kernel,
        out_type=jax.ShapeDtypeStruct((NCORES, NPAD, d_eff), jnp.float32),
        mesh=mesh,
        scratch_types=[pltpu.VMEM_SHARED((NPAD, d_eff), jnp.float32),
                       pltpu.VMEM((2 * NJ, CHUNK), jnp.int32),
                       pltpu.VMEM((2 * NJ, CHUNK), jnp.int32)]
                      + [pltpu.VMEM((CHUNK, d_eff), jnp.float32)
                         for _ in range(NJ)]
                      + [pltpu.SemaphoreType.DMA,
                         pltpu.SemaphoreType.DMA,
                         pltpu.SemaphoreType.DMA],
        compiler_params=pltpu.CompilerParams(use_tc_tiling_on_sc=False),
    )
    def seg_kernel(ta_h, tb_h, sd_h, out_h, acc_s, idx0, idx1, *rest):
        rows = rest[:NJ]
        isem, gsem, ssem = rest[NJ], rest[NJ + 1], rest[NJ + 2]
        c = lax.axis_index("c")
        s = lax.axis_index("s")

        zero16 = jnp.zeros((LANES,), jnp.float32)

        @pl.loop(0, CHUNK)
        def _(r):
            for k in range(d_eff // LANES):
                rows[0][r, pl.ds(k * LANES, LANES)] = zero16
                rows[1][r, pl.ds(k * LANES, LANES)] = zero16

        r0 = s * stripe
        zh = []
        for i in range(ZFULL):
            zh.append(pltpu.async_copy(
                rows[i % 2], acc_s.at[pl.ds(r0 + i * CHUNK, CHUNK)], gsem))
        zh.append(pltpu.async_copy(
            rows[0].at[pl.ds(0, ZTAIL)],
            acc_s.at[pl.ds(r0 + ZFULL * CHUNK, ZTAIL)], ssem))
        for h in zh:
            h.wait()

        plsc.subcore_barrier()

        if full_range:
            base = s * PT
        else:
            base = c * (TR // 2) + s * PT
        sdbase = 2 * base

        def load_block(buf, b):
            return pltpu.async_copy(
                sd_h.at[pl.ds(sdbase + 2 * NJ * b, 2 * NJ)], buf, isem)

        def main_loop(table_h):
            def process(buf, prefetch):
                gh = [pltpu.async_copy(table_h.at[buf.at[2 * k]], rows[k],
                                       gsem)
                      for k in range(NJ)]
                ph = prefetch() if prefetch is not None else None
                sh = []
                for k in range(NJ):
                    gh[k].wait()
                    sh.append(pltpu.async_copy(
                        rows[k], acc_s.at[buf.at[2 * k + 1]], ssem, add=True))
                for h in sh:
                    h.wait()
                if ph is not None:
                    ph.wait()

            load_block(idx0, 0).wait()

            @pl.loop(0, PAIRS)
            def _(o2):
                b0 = 2 * o2
                process(idx0, lambda: load_block(idx1, b0 + 1))
                nxt = lax.min(b0 + 2, NB - 1)
                process(idx1, lambda: load_block(idx0, nxt))

            if TAIL:
                process(idx0, None)

        if table_a is table_b:
            main_loop(ta_h)
        else:
            @pl.when(c == 0)
            def _():
                main_loop(ta_h)

            @pl.when(c == 1)
            def _():
                main_loop(tb_h)

        plsc.subcore_barrier()
        pltpu.sync_copy(acc_s.at[pl.ds(r0, stripe)],
                        out_h.at[c, pl.ds(r0, stripe)])

    return seg_kernel(table_a, table_b, sd)


def _elu(x):
    return jnp.where(x > 0, x, jnp.expm1(x))


HBLK = 400


def _head_tc(ha, hb, p0, p1, w3a, w3b, wna, wnb, b3, w1, b1, w2p, b2p):
    def body(ha_r, hb_r, p0_r, p1_r, w3a_r, w3b_r, wna_r, wnb_r, b3_r,
             w1_r, b1_r, w2_r, b2_r, o_r):
        f32 = jnp.float32
        t = (jnp.dot(ha_r[...], w3a_r[...], preferred_element_type=f32)
             + jnp.dot(hb_r[...], w3b_r[...], preferred_element_type=f32)
             + jnp.dot(p0_r[...], wna_r[...], preferred_element_type=f32)
             + jnp.dot(p1_r[...], wnb_r[...], preferred_element_type=f32)
             + b3_r[...])
        t = jnp.where(t > 0, t, jnp.exp(jnp.minimum(t, 0.)) - 1.)
        u = jnp.dot(t, w1_r[...], preferred_element_type=f32) + b1_r[...]
        u = jnp.where(u > 0, u, jnp.exp(jnp.minimum(u, 0.)) - 1.)
        v = jnp.dot(u, w2_r[...], preferred_element_type=f32) + b2_r[...]
        m = jnp.max(v, axis=1, keepdims=True)
        lse = jnp.log(jnp.sum(jnp.exp(v - m), axis=1, keepdims=True)) + m
        o_r[...] = v - lse

    row = lambda i: (i, 0)
    fix = lambda i: (0, 0)
    return pl.pallas_call(
        body,
        grid=(N_NODES // HBLK,),
        in_specs=[pl.BlockSpec((HBLK, 32), row)] * 4
                 + [pl.BlockSpec((32, 128), fix)] * 4
                 + [pl.BlockSpec((1, 128), fix),
                    pl.BlockSpec((128, 256), fix),
                    pl.BlockSpec((1, 256), fix),
                    pl.BlockSpec((256, 128), fix),
                    pl.BlockSpec((1, 128), fix)],
        out_specs=pl.BlockSpec((HBLK, 128), row),
        out_shape=jax.ShapeDtypeStruct((N_NODES, 128), jnp.float32),
    )(ha, hb, p0, p1, w3a, w3b, wna, wnb, b3, w1, b1, w2p, b2p)


def kernel(x, edge_index, batch, fc0_W, fc0_b, Wr1, Wn1, b1, Wr2, Wn2, b2,
           Wr3, Wn3, b3, fc1_W, fc1_b, fc2_W, fc2_b):
    pad = EP - EDGES
    src = edge_index[0]
    dst = edge_index[1]
    pad_iota = jnp.arange(pad, dtype=jnp.int32)
    src_p = jnp.concatenate([src, pad_iota % N_NODES])
    dst_p = jnp.concatenate([dst, N_NODES + pad_iota % (NPAD - N_NODES)])

    sd_e = jnp.stack([src_p.reshape(EP // CHUNK, CHUNK),
                      dst_p.reshape(EP // CHUNK, CHUNK)],
                     axis=1).reshape(2 * EP // CHUNK, CHUNK)

    h = _elu(x @ fc0_W + fc0_b)

    p = _segsum_sc(h, h, sd_e, 16, False)
    agg = (p[0] + p[1])[:N_NODES]
    h = _elu(h @ Wr1 + agg @ Wn1 + b1)

    p = _segsum_sc(h, h, sd_e, 32, False)
    agg = (p[0] + p[1])[:N_NODES]
    ha = _elu(h @ Wr2[:, :32] + agg @ Wn2[:, :32] + b2[:32])
    hb = _elu(h @ Wr2[:, 32:] + agg @ Wn2[:, 32:] + b2[32:])

    p = _segsum_sc(ha, hb, sd_e, 32, True)

    fc2p = jnp.zeros((256, 128), jnp.float32).at[:, :NUM_CLS].set(fc2_W)
    fc2bp = jnp.full((1, 128), -1e30, jnp.float32).at[0, :NUM_CLS].set(fc2_b)
    out = _head_tc(ha, hb, p[0], p[1], Wr3[:32], Wr3[32:], Wn3[:32],
                   Wn3[32:], b3.reshape(1, 128), fc1_W,
                   fc1_b.reshape(1, 256), fc2p, fc2bp)
    return out[:, :NUM_CLS]


# speedup vs baseline: 1.0988x; 1.0988x over previous
import functools

import jax
import jax.numpy as jnp
from jax import lax
from jax.experimental import pallas as pl
from jax.experimental.pallas import tpu as pltpu
from jax.experimental.pallas import tpu_sc as plsc

N_NODES = 50000
NPAD = 50048
EDGES = 800000
EP = 819200
LANES = 16
NCORES = 2
NSUB = 16
CHUNK = 128
NJ = 4
IDXG = 8
NUM_CLS = 40


def _segsum_sc(table_a, table_b, srcs, dsts, d_eff, full_range):
    TR = srcs.shape[0]
    PT = TR // NSUB if full_range else TR // (NCORES * NSUB)
    NB = PT // IDXG
    assert PT % IDXG == 0
    PAIRS, TAIL = NB // 2, NB % 2
    stripe = NPAD // NSUB
    ZFULL, ZTAIL = stripe // CHUNK, stripe % CHUNK
    assert ZTAIL % 8 == 0
    mesh = plsc.VectorSubcoreMesh(core_axis_name="c", subcore_axis_name="s")

    @functools.partial(
        pl.kernel,
        out_type=jax.ShapeDtypeStruct((NCORES, NPAD, d_eff), jnp.float32),
        mesh=mesh,
        scratch_types=[pltpu.VMEM_SHARED((NPAD, d_eff), jnp.float32)]
                      + [pltpu.VMEM((IDXG, CHUNK), jnp.int32)
                         for _ in range(4)]
                      + [pltpu.VMEM((CHUNK, d_eff), jnp.float32)
                         for _ in range(NJ)]
                      + [pltpu.SemaphoreType.DMA,
                         pltpu.SemaphoreType.DMA,
                         pltpu.SemaphoreType.DMA],
        compiler_params=pltpu.CompilerParams(use_tc_tiling_on_sc=False),
    )
    def seg_kernel(ta_h, tb_h, srcs_h, dsts_h, out_h, acc_s,
                   s0, d0, s1, d1, *rest):
        rows = rest[:NJ]
        isem, gsem, ssem = rest[NJ], rest[NJ + 1], rest[NJ + 2]
        c = lax.axis_index("c")
        s = lax.axis_index("s")

        zero16 = jnp.zeros((LANES,), jnp.float32)

        @pl.loop(0, CHUNK)
        def _(r):
            for k in range(d_eff // LANES):
                rows[0][r, pl.ds(k * LANES, LANES)] = zero16
                rows[1][r, pl.ds(k * LANES, LANES)] = zero16

        r0 = s * stripe
        zh = []
        for i in range(ZFULL):
            zh.append(pltpu.async_copy(
                rows[i % 2], acc_s.at[pl.ds(r0 + i * CHUNK, CHUNK)], gsem))
        if ZTAIL:
            zh.append(pltpu.async_copy(
                rows[0].at[pl.ds(0, ZTAIL)],
                acc_s.at[pl.ds(r0 + ZFULL * CHUNK, ZTAIL)], ssem))
        for h in zh:
            h.wait()

        plsc.subcore_barrier()

        if full_range:
            base = s * PT
        else:
            base = c * (TR // 2) + s * PT

        def load_block(sbuf, dbuf, b):
            off = base + IDXG * b
            return (pltpu.async_copy(srcs_h.at[pl.ds(off, IDXG)], sbuf, isem),
                    pltpu.async_copy(dsts_h.at[pl.ds(off, IDXG)], dbuf, isem))

        def main_loop(table_h):
            def process(sbuf, dbuf, prefetch):
                gh = [pltpu.async_copy(table_h.at[sbuf.at[k]], rows[k], gsem)
                      for k in range(NJ)]
                ph = prefetch() if prefetch is not None else None
                sh = []
                for k in range(NJ):
                    gh[k].wait()
                    sh.append(pltpu.async_copy(
                        rows[k], acc_s.at[dbuf.at[k]], ssem, add=True))
                gh2 = []
                for k in range(NJ):
                    sh[k].wait()
                    gh2.append(pltpu.async_copy(
                        table_h.at[sbuf.at[NJ + k]], rows[k], gsem))
                sh2 = []
                for k in range(NJ):
                    gh2[k].wait()
                    sh2.append(pltpu.async_copy(
                        rows[k], acc_s.at[dbuf.at[NJ + k]], ssem, add=True))
                for h in sh2:
                    h.wait()
                if ph is not None:
                    ph[0].wait()
                    ph[1].wait()

            for h in load_block(s0, d0, 0):
                h.wait()

            @pl.loop(0, PAIRS)
            def _(o2):
                b0 = 2 * o2
                process(s0, d0, lambda: load_block(s1, d1, b0 + 1))
                nxt = lax.min(b0 + 2, NB - 1)
                process(s1, d1, lambda: load_block(s0, d0, nxt))

            if TAIL:
                process(s0, d0, None)

        if table_a is table_b:
            main_loop(ta_h)
        else:
            @pl.when(c == 0)
            def _():
                main_loop(ta_h)

            @pl.when(c == 1)
            def _():
                main_loop(tb_h)

        plsc.subcore_barrier()
        pltpu.sync_copy(acc_s.at[pl.ds(r0, stripe)],
                        out_h.at[c, pl.ds(r0, stripe)])

    return seg_kernel(table_a, table_b, srcs, dsts)


def _elu(x):
    return jnp.where(x > 0, x, jnp.expm1(x))


def kernel(x, edge_index, batch, fc0_W, fc0_b, Wr1, Wn1, b1, Wr2, Wn2, b2,
           Wr3, Wn3, b3, fc1_W, fc1_b, fc2_W, fc2_b):
    pad = EP - EDGES
    src = edge_index[0]
    dst = edge_index[1]
    pad_iota = jnp.arange(pad, dtype=jnp.int32)
    src_p = jnp.concatenate([src, pad_iota % N_NODES])
    dst_p = jnp.concatenate([dst, N_NODES + pad_iota % (NPAD - N_NODES)])

    srcs_e = src_p.reshape(EP // CHUNK, CHUNK)
    dsts_e = dst_p.reshape(EP // CHUNK, CHUNK)

    h = _elu(x @ fc0_W + fc0_b)

    p = _segsum_sc(h, h, srcs_e, dsts_e, 16, False)
    agg = (p[0] + p[1])[:N_NODES]
    h = _elu(h @ Wr1 + agg @ Wn1 + b1)

    p = _segsum_sc(h, h, srcs_e, dsts_e, 32, False)
    agg = (p[0] + p[1])[:N_NODES]
    ha = _elu(h @ Wr2[:, :32] + agg @ Wn2[:, :32] + b2[:32])
    hb = _elu(h @ Wr2[:, 32:] + agg @ Wn2[:, 32:] + b2[32:])

    p = _segsum_sc(ha, hb, srcs_e, dsts_e, 32, True)
    h = _elu(ha @ Wr3[:32] + hb @ Wr3[32:]
             + p[0][:N_NODES] @ Wn3[:32] + p[1][:N_NODES] @ Wn3[32:]
             + b3)

    h = _elu(h @ fc1_W + fc1_b)
    h = h @ fc2_W + fc2_b
    return jax.nn.log_softmax(h, axis=1)
